# baseline (speedup 1.0000x reference)
import jax
import jax.numpy as jnp
from jax import lax
from jax.experimental import pallas as pl
from jax.experimental.pallas import tpu as pltpu
from jax.experimental.pallas import tpu_sc as plsc

B = 4096
L = 50
EMB = 64
NC, NS = 2, 16
NW = NC * NS
SEQ_W = B // NW
NBUF = 5
AHEAD = 2


def _emb_kernel(src_tbl, tgt_tbl, src_tok, tgt_tok, out,
                idx_v, idx_t, *scratch):
    rows = scratch[:NBUF]
    rows_t = scratch[NBUF:2 * NBUF]
    gsem = scratch[2 * NBUF:3 * NBUF]
    wsem = scratch[3 * NBUF:]
    wid = lax.axis_index("s") * NC + lax.axis_index("c")
    b0 = wid * SEQ_W
    lanes = lax.iota(jnp.int32, 16)

    for t, (tbl, tok) in enumerate(((src_tbl, src_tok), (tgt_tbl, tgt_tok))):
        pltpu.sync_copy(tok.at[pl.ds(b0 * L, SEQ_W * L)], idx_v)

        @pl.loop(0, L)
        def _(l):
            for k in range(SEQ_W // 16):
                src = (lanes + 16 * k) * L + l
                idx_t[pl.ds(l * SEQ_W + 16 * k, 16)] = plsc.load_gather(
                    idx_v, [src])

        def gather(l, b):
            pltpu.async_copy(tbl.at[idx_t.at[pl.ds(l * SEQ_W, SEQ_W)]],
                             rows[b], gsem[b])

        def gather_wait(l, b):
            pltpu.make_async_copy(tbl.at[idx_t.at[pl.ds(l * SEQ_W, SEQ_W)]],
                                  rows[b], gsem[b]).wait()

        def wb(l, b):
            pltpu.async_copy(rows_t[b], out.at[t, l, :, pl.ds(b0, SEQ_W)],
                             wsem[b])

        def wb_wait(l, b):
            pltpu.make_async_copy(rows_t[b],
                                  out.at[t, l, :, pl.ds(b0, SEQ_W)],
                                  wsem[b]).wait()

        def transpose(b):
            src = rows[b]
            dst = rows_t[b]

            @pl.loop(0, EMB)
            def _(e):
                col = jnp.full((16,), e, jnp.int32)
                for j in range(SEQ_W // 16):
                    dst[e, pl.ds(16 * j, 16)] = plsc.load_gather(
                        src, [lanes + 16 * j, col])

        for b in range(AHEAD):
            gather(b, b)

        @pl.loop(0, L, step=NBUF)
        def _(l0):
            for b in range(NBUF):
                l = l0 + b
                nxt = (b + AHEAD) % NBUF

                @pl.when(l < L - AHEAD)
                def _():
                    gather(l + AHEAD, nxt)

                gather_wait(l, b)

                @pl.when(l >= NBUF)
                def _():
                    wb_wait(l - NBUF, b)

                transpose(b)
                wb(l, b)

        for l in range(L - NBUF, L):
            wb_wait(l, l % NBUF)


@jax.jit
def kernel(src_tokens, tgt_tokens, src_table, tgt_table):
    mesh = plsc.VectorSubcoreMesh(core_axis_name="c", subcore_axis_name="s")
    y = pl.kernel(
        _emb_kernel,
        out_type=jax.ShapeDtypeStruct((2, L, EMB, B), jnp.float32),
        mesh=mesh,
        scratch_types=(
            [pltpu.VMEM((SEQ_W * L,), jnp.int32),
             pltpu.VMEM((SEQ_W * L,), jnp.int32)]
            + [pltpu.VMEM((SEQ_W, EMB), jnp.float32) for _ in range(NBUF)]
            + [pltpu.VMEM((EMB, SEQ_W), jnp.float32) for _ in range(NBUF)]
            + [pltpu.SemaphoreType.DMA for _ in range(2 * NBUF)]
        ),
        compiler_params=pltpu.CompilerParams(use_tc_tiling_on_sc=False,
                                             needs_layout_passes=False),
    )(src_table, tgt_table, src_tokens.reshape(B * L).astype(jnp.int32),
      tgt_tokens.reshape(B * L).astype(jnp.int32))
    return jnp.transpose(y, (0, 3, 1, 2))


# speedup vs baseline: 1.9639x; 1.9639x over previous
import jax
import jax.numpy as jnp
from jax import lax
from jax.experimental import pallas as pl
from jax.experimental.pallas import tpu as pltpu
from jax.experimental.pallas import tpu_sc as plsc

B = 4096
L = 50
EMB = 64
NC, NS = 2, 16
NW = NC * NS
SEQ_W = B // NW
NBUF = 5
AHEAD = 2


def _emb_kernel(src_tbl, tgt_tbl, src_tok, tgt_tok, out,
                idx_v, idx_t, *scratch):
    rows = scratch[:NBUF]
    gsem = scratch[NBUF:2 * NBUF]
    wsem = scratch[2 * NBUF:]
    wid = lax.axis_index("s") * NC + lax.axis_index("c")
    b0 = wid * SEQ_W
    lanes = lax.iota(jnp.int32, 16)

    for t, (tbl, tok) in enumerate(((src_tbl, src_tok), (tgt_tbl, tgt_tok))):
        pltpu.sync_copy(tok.at[pl.ds(b0 * L, SEQ_W * L)], idx_v)

        @pl.loop(0, L)
        def _(l):
            for k in range(SEQ_W // 16):
                src = (lanes + 16 * k) * L + l
                idx_t[pl.ds(l * SEQ_W + 16 * k, 16)] = plsc.load_gather(
                    idx_v, [src])

        def gather(l, b):
            pltpu.async_copy(tbl.at[idx_t.at[pl.ds(l * SEQ_W, SEQ_W)]],
                             rows[b], gsem[b])

        def gather_wait(l, b):
            pltpu.make_async_copy(tbl.at[idx_t.at[pl.ds(l * SEQ_W, SEQ_W)]],
                                  rows[b], gsem[b]).wait()

        def wb(l, b):
            pltpu.async_copy(rows[b], out.at[t, l, pl.ds(b0, SEQ_W)],
                             wsem[b])

        def wb_wait(l, b):
            pltpu.make_async_copy(rows[b], out.at[t, l, pl.ds(b0, SEQ_W)],
                                  wsem[b]).wait()

        for b in range(AHEAD):
            gather(b, b)

        @pl.loop(0, L, step=NBUF)
        def _(l0):
            for b in range(NBUF):
                l = l0 + b
                nxt = (b + AHEAD) % NBUF

                @pl.when(l >= NBUF - AHEAD)
                def _():
                    wb_wait(l + AHEAD - NBUF, nxt)

                @pl.when(l < L - AHEAD)
                def _():
                    gather(l + AHEAD, nxt)

                gather_wait(l, b)
                wb(l, b)

        for l in range(L - (NBUF - AHEAD), L):
            wb_wait(l, l % NBUF)


@jax.jit
def kernel(src_tokens, tgt_tokens, src_table, tgt_table):
    mesh = plsc.VectorSubcoreMesh(core_axis_name="c", subcore_axis_name="s")
    y = pl.kernel(
        _emb_kernel,
        out_type=jax.ShapeDtypeStruct((2, L, B, EMB), jnp.float32),
        mesh=mesh,
        scratch_types=(
            [pltpu.VMEM((SEQ_W * L,), jnp.int32),
             pltpu.VMEM((SEQ_W * L,), jnp.int32)]
            + [pltpu.VMEM((SEQ_W, EMB), jnp.float32) for _ in range(NBUF)]
            + [pltpu.SemaphoreType.DMA for _ in range(2 * NBUF)]
        ),
        compiler_params=pltpu.CompilerParams(use_tc_tiling_on_sc=False,
                                             needs_layout_passes=False),
    )(src_table, tgt_table, src_tokens.reshape(B * L).astype(jnp.int32),
      tgt_tokens.reshape(B * L).astype(jnp.int32))
    return jnp.transpose(y, (0, 2, 1, 3))
